# Initial kernel scaffold; baseline (speedup 1.0000x reference)
import jax
import jax.numpy as jnp
from jax.experimental import pallas as pl


def kernel(rel_idx, src_idx, edge_batch, edge_src, edge_rel, edge_dst, edge_ts, rel_tables, W1, W2, W_past, Wc, bc, week, month, season):
    raise NotImplementedError("write your pallas kernel here")


# speedup vs baseline: 1.8668x; 1.8668x over previous
import functools

import jax
import jax.numpy as jnp
from jax.experimental import pallas as pl

B = 8
N_ENT = 10000
E = 160000
H = 128
ATTN = 64
NL = 2
NPAD = 80256
BK = 1000
BN = 1056


def _edge_math(idx, TA, TP, W2r, gp, ga):
    rel = idx[:, 0:1]
    b = idx[:, 1:2]
    wk = idx[:, 2:3]
    mo = idx[:, 3:4]
    se = idx[:, 4:5]
    colA = jax.lax.broadcasted_iota(jnp.int32, (BK, 256), 1)
    ohA = ((colA == rel) | (colA == 232 + b)).astype(jnp.float32)
    a_pre = jnp.dot(ohA, TA, preferred_element_type=jnp.float32)
    if ga is not None:
        a_pre = a_pre + ga
    att = jax.nn.sigmoid(
        jnp.dot(jax.nn.relu(a_pre), W2r, preferred_element_type=jnp.float32))
    colP = jax.lax.broadcasted_iota(jnp.int32, (BK, 512), 1)
    ohP = ((colP == rel) | (colP == 232 + wk) | (colP == 239 + mo)
           | (colP == 269 + se)).astype(jnp.float32)
    base = jnp.dot(ohP, TP, preferred_element_type=jnp.float32)
    if gp is not None:
        base = base + gp
    return att * base


def _edge_kernel0(idx_ref, ta_ref, tp_ref, w2_ref, out_ref):
    out_ref[...] = _edge_math(idx_ref[...], ta_ref[...], tp_ref[...],
                              w2_ref[...], None, None)


def _edge_kernel1(idx_ref, ta_ref, tp_ref, w2_ref, gp_ref, ga_ref, out_ref):
    out_ref[...] = _edge_math(idx_ref[...], ta_ref[...], tp_ref[...],
                              w2_ref[...], gp_ref[...], ga_ref[...])


def _edge_call(idx, TA, TP, W2r, gp=None, ga=None):
    nb = E // BK
    full = lambda shape: pl.BlockSpec(shape, lambda i: (0, 0))
    in_specs = [
        pl.BlockSpec((BK, 8), lambda i: (i, 0)),
        full((256, ATTN)),
        full((512, H)),
        full((ATTN, H)),
    ]
    args = [idx, TA, TP, W2r]
    kern = _edge_kernel0
    if gp is not None:
        in_specs += [pl.BlockSpec((BK, H), lambda i: (i, 0)),
                     pl.BlockSpec((BK, ATTN), lambda i: (i, 0))]
        args += [gp, ga]
        kern = _edge_kernel1
    return pl.pallas_call(
        kern,
        grid=(nb,),
        in_specs=in_specs,
        out_specs=pl.BlockSpec((BK, H), lambda i: (i, 0)),
        out_shape=jax.ShapeDtypeStruct((E, H), jnp.float32),
    )(*args)


def _node_kernel(h_ref, wp_ref, wh_ref, gp_ref, ga_ref):
    h = jax.nn.relu(h_ref[...])
    gp_ref[...] = jnp.dot(h, wp_ref[...], preferred_element_type=jnp.float32)
    ga_ref[...] = jnp.dot(h, wh_ref[...], preferred_element_type=jnp.float32)


def _node_call(h_raw, W_past, W1h):
    nb = NPAD // BN
    return pl.pallas_call(
        _node_kernel,
        grid=(nb,),
        in_specs=[
            pl.BlockSpec((BN, H), lambda i: (i, 0)),
            pl.BlockSpec((H, H), lambda i: (0, 0)),
            pl.BlockSpec((H, ATTN), lambda i: (0, 0)),
        ],
        out_specs=[pl.BlockSpec((BN, H), lambda i: (i, 0)),
                   pl.BlockSpec((BN, ATTN), lambda i: (i, 0))],
        out_shape=[jax.ShapeDtypeStruct((NPAD, H), jnp.float32),
                   jax.ShapeDtypeStruct((NPAD, ATTN), jnp.float32)],
    )(h_raw, W_past, W1h)


def _final_kernel(h_ref, wc_ref, bc_ref, res_ref, prob_ref):
    h = jax.nn.relu(h_ref[...])
    y = jnp.dot(h, wc_ref[...], preferred_element_type=jnp.float32)
    y = y + bc_ref[0:1, :]
    m = jnp.max(y)
    e = jnp.exp(y - m)
    s = jnp.sum(e) / float(H)
    res_ref[...] = y[:, 0:1]
    prob_ref[...] = (e / s)[:, 0:1]


def _final_call(h2_raw, WcRep, bc_row):
    return pl.pallas_call(
        _final_kernel,
        grid=(B,),
        in_specs=[
            pl.BlockSpec((N_ENT, H), lambda i: (i, 0)),
            pl.BlockSpec((H, H), lambda i: (0, 0)),
            pl.BlockSpec((8, H), lambda i: (0, 0)),
        ],
        out_specs=[pl.BlockSpec((N_ENT, 1), lambda i: (i, 0)),
                   pl.BlockSpec((N_ENT, 1), lambda i: (i, 0))],
        out_shape=[jax.ShapeDtypeStruct((B * N_ENT, 1), jnp.float32),
                   jax.ShapeDtypeStruct((B * N_ENT, 1), jnp.float32)],
    )(h2_raw, WcRep, bc_row)


def kernel(rel_idx, src_idx, edge_batch, edge_src, edge_rel, edge_dst,
           edge_ts, rel_tables, W1, W2, W_past, Wc, bc, week, month, season):
    i32 = jnp.int32
    rel_idx = rel_idx.astype(i32)
    edge_batch = edge_batch.astype(i32)
    edge_src = edge_src.astype(i32)
    edge_rel = edge_rel.astype(i32)
    edge_dst = edge_dst.astype(i32)
    edge_ts = edge_ts.astype(i32)

    t = edge_ts // 24
    wk = t % 7
    mo = t % 30
    se = t % 120
    src_gid = edge_batch * N_ENT + edge_src
    dst_gid = edge_batch * N_ENT + edge_dst

    idx_pack = jnp.stack(
        [edge_rel, edge_batch, wk, mo, se, src_gid, dst_gid, edge_rel],
        axis=1)

    weekP = week @ W_past
    monthP = month @ W_past
    seasonP = season @ W_past
    TAs, TPs, W2rs, W1hs = [], [], [], []
    for i in range(NL):
        RT = rel_tables[i]
        relA = RT @ W1[i][H:2 * H]
        qA = RT[rel_idx] @ W1[i][2 * H:]
        TA = jnp.concatenate(
            [relA, qA, jnp.zeros((256 - 232 - B, ATTN), jnp.float32)], axis=0)
        relP = RT @ W_past
        TP = jnp.concatenate(
            [relP, weekP, monthP, seasonP,
             jnp.zeros((512 - 389, H), jnp.float32)], axis=0)
        TAs.append(TA)
        TPs.append(TP)
        W2rs.append(jnp.tile(W2[i], (1, H)))
        W1hs.append(W1[i][:H])

    msg0 = _edge_call(idx_pack, TAs[0], TPs[0], W2rs[0])
    h1_raw = jax.ops.segment_sum(msg0, dst_gid, num_segments=NPAD)

    Gp1, Ga1 = _node_call(h1_raw, W_past, W1hs[1])

    gp = Gp1[src_gid]
    ga = Ga1[src_gid]
    msg1 = _edge_call(idx_pack, TAs[1], TPs[1], W2rs[1], gp, ga)
    h2_raw = jax.ops.segment_sum(msg1, dst_gid, num_segments=NPAD)

    res_col, prob_col = _final_call(h2_raw[:B * N_ENT], jnp.tile(Wc, (1, H)),
                                    jnp.tile(bc.reshape(1, 1), (8, H)))
    result = res_col.reshape(B, N_ENT)
    probs = prob_col.reshape(B, N_ENT)
    return (result, probs)
